# baseline (speedup 1.0000x reference)
import functools

import jax
import jax.numpy as jnp
from jax import lax
from jax.experimental import pallas as pl
from jax.experimental.pallas import tpu as pltpu
from jax.experimental.pallas import tpu_sc as plsc

NUM_EMB = 8192
EMB_DIM = 256
B = 16
HW = 1024
TN = 2736
N_BLOCKS = 3

_NC = 2
_NS = 16
_NW = _NC * _NS
_ROWS_PER_W = (B * HW) // _NW
_CHUNK = 128
_NCHUNK = _ROWS_PER_W // _CHUNK


def _phase1_body(z_ref, emb_ref, zn_ref, val_ref, idx_ref, tru_ref):
    j = pl.program_id(1)
    e2 = emb_ref[...]
    z = z_ref[0]
    g2 = lax.dot_general(e2.astype(jnp.bfloat16), z.astype(jnp.bfloat16),
                         (((1,), (0,)), ((), ())),
                         preferred_element_type=jnp.float32)
    en = 0.25 * jnp.sum(e2 * e2, axis=1, keepdims=True)
    dist = (zn_ref[0] + en) + g2
    rows = lax.broadcasted_iota(jnp.int32, dist.shape, 0) + j * TN
    dist = lax.cond(
        j == N_BLOCKS - 1,
        lambda d: jnp.where(rows < NUM_EMB, d, jnp.float32(jnp.inf)),
        lambda d: d,
        dist)
    bmin = jnp.min(dist, axis=0, keepdims=True)
    tie = jnp.where(dist == bmin, rows, jnp.int32(2147483647))
    barg = jnp.min(tie, axis=0, keepdims=True)
    bq = bmin.astype(jnp.bfloat16).astype(jnp.float32)

    @pl.when(j == 0)
    def _():
        val_ref[0] = bq
        idx_ref[0] = barg
        tru_ref[0] = bmin

    @pl.when(j != 0)
    def _():
        cur = val_ref[0]
        better = bmin < cur
        val_ref[0] = jnp.where(better, bq, cur)
        idx_ref[0] = jnp.where(better, barg, idx_ref[0])
        tru_ref[0] = jnp.where(better, bmin, tru_ref[0])


def _phase1(z3, emb_weight, zn3, interpret=False):
    return pl.pallas_call(
        _phase1_body,
        grid=(B, N_BLOCKS),
        in_specs=[
            pl.BlockSpec((1, EMB_DIM, HW), lambda b, j: (b, 0, 0)),
            pl.BlockSpec((TN, EMB_DIM), lambda b, j: (j, 0)),
            pl.BlockSpec((1, 1, HW), lambda b, j: (b, 0, 0)),
        ],
        out_specs=[
            pl.BlockSpec((1, 1, HW), lambda b, j: (b, 0, 0)),
            pl.BlockSpec((1, 1, HW), lambda b, j: (b, 0, 0)),
            pl.BlockSpec((1, 1, HW), lambda b, j: (b, 0, 0)),
        ],
        out_shape=[
            jax.ShapeDtypeStruct((B, 1, HW), jnp.float32),
            jax.ShapeDtypeStruct((B, 1, HW), jnp.int32),
            jax.ShapeDtypeStruct((B, 1, HW), jnp.float32),
        ],
        compiler_params=pltpu.CompilerParams(
            dimension_semantics=("parallel", "arbitrary")),
        interpret=interpret,
    )(z3, emb_weight, zn3)


def _sc_gather_body(table_hbm, idx_hbm, out_hbm, idx_v, rows_v, sem):
    wid = lax.axis_index("s") * _NC + lax.axis_index("c")
    for k in range(_NCHUNK):
        base = wid * _ROWS_PER_W + k * _CHUNK
        pltpu.sync_copy(idx_hbm.at[pl.ds(base, _CHUNK)], idx_v)
        pltpu.async_copy(table_hbm.at[idx_v], rows_v, sem).wait()
        pltpu.sync_copy(rows_v, out_hbm.at[pl.ds(base, _CHUNK)])


@functools.cache
def _make_sc_gather():
    return functools.partial(
        pl.kernel,
        mesh=plsc.VectorSubcoreMesh(core_axis_name="c", subcore_axis_name="s"),
        out_type=jax.ShapeDtypeStruct((B * HW, EMB_DIM), jnp.float32),
        scratch_types=[
            pltpu.VMEM((_CHUNK,), jnp.int32),
            pltpu.VMEM((_CHUNK, EMB_DIM), jnp.float32),
            pltpu.SemaphoreType.DMA,
        ],
    )(_sc_gather_body)


def kernel(z_e, emb_weight):
    b, c, h, w = z_e.shape
    z_flat = jnp.transpose(z_e, (0, 2, 3, 1)).reshape(b * h * w, c)
    zn = jnp.sum(z_flat ** 2, axis=-1, keepdims=True)
    zn3 = zn.reshape(b, 1, h * w)
    z3 = z_e.reshape(b, c, h * w)

    val3, idx3, tru3 = _phase1(z3, emb_weight * jnp.float32(-2.0), zn3)
    enc_idx = idx3.reshape(b * h * w)

    m = jnp.sum(tru3) / (b * h * w * c)
    loss = m + 0.25 * m

    zq_flat = _make_sc_gather()(emb_weight, enc_idx)
    z_q_st = jnp.transpose(zq_flat.reshape(b, h, w, c), (0, 3, 1, 2))
    return (z_q_st, loss, enc_idx)


# speedup vs baseline: 1.6100x; 1.6100x over previous
import functools

import jax
import jax.numpy as jnp
from jax import lax
from jax.experimental import pallas as pl
from jax.experimental.pallas import tpu as pltpu
from jax.experimental.pallas import tpu_sc as plsc

NUM_EMB = 8192
EMB_DIM = 256
B = 16
HW = 1024
TN = 2736
N_BLOCKS = 3

_NC = 2
_NS = 16
_NW = _NC * _NS
_ROWS_PER_W = (B * HW) // _NW
_CHUNK = 128
_NCHUNK = _ROWS_PER_W // _CHUNK


def _phase1_body(z_ref, emb_ref, zn_ref, val_ref, idx_ref, tru_ref):
    j = pl.program_id(1)
    e2 = emb_ref[...]
    z = z_ref[0]
    g2 = lax.dot_general(e2.astype(jnp.bfloat16), z.astype(jnp.bfloat16),
                         (((1,), (0,)), ((), ())),
                         preferred_element_type=jnp.float32)
    en = 0.25 * jnp.sum(e2 * e2, axis=1, keepdims=True)
    dist = (zn_ref[0] + en) + g2
    rows = lax.broadcasted_iota(jnp.int32, dist.shape, 0) + j * TN
    dist = jnp.where(rows < NUM_EMB, dist, jnp.float32(jnp.inf))
    bmin = jnp.min(dist, axis=0, keepdims=True)
    tie = jnp.where(dist == bmin, rows, jnp.int32(2147483647))
    barg = jnp.min(tie, axis=0, keepdims=True)
    bq = bmin.astype(jnp.bfloat16).astype(jnp.float32)

    @pl.when(j == 0)
    def _():
        val_ref[0] = bq
        idx_ref[0] = barg
        tru_ref[0] = bmin

    @pl.when(j != 0)
    def _():
        cur = val_ref[0]
        better = bmin < cur
        val_ref[0] = jnp.where(better, bq, cur)
        idx_ref[0] = jnp.where(better, barg, idx_ref[0])
        tru_ref[0] = jnp.where(better, bmin, tru_ref[0])


def _phase1(z3, emb_weight, zn3, interpret=False):
    return pl.pallas_call(
        _phase1_body,
        grid=(B, N_BLOCKS),
        in_specs=[
            pl.BlockSpec((1, EMB_DIM, HW), lambda b, j: (b, 0, 0)),
            pl.BlockSpec((TN, EMB_DIM), lambda b, j: (j, 0)),
            pl.BlockSpec((1, 1, HW), lambda b, j: (b, 0, 0)),
        ],
        out_specs=[
            pl.BlockSpec((1, 1, HW), lambda b, j: (b, 0, 0)),
            pl.BlockSpec((1, 1, HW), lambda b, j: (b, 0, 0)),
            pl.BlockSpec((1, 1, HW), lambda b, j: (b, 0, 0)),
        ],
        out_shape=[
            jax.ShapeDtypeStruct((B, 1, HW), jnp.float32),
            jax.ShapeDtypeStruct((B, 1, HW), jnp.int32),
            jax.ShapeDtypeStruct((B, 1, HW), jnp.float32),
        ],
        compiler_params=pltpu.CompilerParams(
            dimension_semantics=("parallel", "arbitrary")),
        interpret=interpret,
    )(z3, emb_weight, zn3)


def _sc_gather_body(table_hbm, idx_hbm, out_hbm, idx_v, rows_v, sem):
    wid = lax.axis_index("s") * _NC + lax.axis_index("c")
    for k in range(_NCHUNK):
        base = wid * _ROWS_PER_W + k * _CHUNK
        pltpu.sync_copy(idx_hbm.at[pl.ds(base, _CHUNK)], idx_v)
        pltpu.async_copy(table_hbm.at[idx_v], rows_v, sem).wait()
        pltpu.sync_copy(rows_v, out_hbm.at[pl.ds(base, _CHUNK)])


@functools.cache
def _make_sc_gather():
    return functools.partial(
        pl.kernel,
        mesh=plsc.VectorSubcoreMesh(core_axis_name="c", subcore_axis_name="s"),
        out_type=jax.ShapeDtypeStruct((B * HW, EMB_DIM), jnp.float32),
        scratch_types=[
            pltpu.VMEM((_CHUNK,), jnp.int32),
            pltpu.VMEM((_CHUNK, EMB_DIM), jnp.float32),
            pltpu.SemaphoreType.DMA,
        ],
    )(_sc_gather_body)


def kernel(z_e, emb_weight):
    b, c, h, w = z_e.shape
    z_flat = jnp.transpose(z_e, (0, 2, 3, 1)).reshape(b * h * w, c)
    zn = jnp.sum(z_flat ** 2, axis=-1, keepdims=True)
    zn3 = zn.reshape(b, 1, h * w)
    z3 = z_e.reshape(b, c, h * w)

    val3, idx3, tru3 = _phase1(z3, emb_weight * jnp.float32(-2.0), zn3)
    enc_idx = idx3.reshape(b * h * w)

    m = jnp.sum(tru3) / (b * h * w * c)
    loss = m + 0.25 * m

    zq_flat = _make_sc_gather()(emb_weight, enc_idx)
    z_q_st = jnp.transpose(zq_flat.reshape(b, h, w, c), (0, 3, 1, 2))
    return (z_q_st, loss, enc_idx)


# speedup vs baseline: 1.7010x; 1.0565x over previous
import functools

import jax
import jax.numpy as jnp
from jax import lax
from jax.experimental import pallas as pl
from jax.experimental.pallas import tpu as pltpu
from jax.experimental.pallas import tpu_sc as plsc

NUM_EMB = 8192
EMB_DIM = 256
B = 16
HW = 1024
TN = 2736
N_BLOCKS = 3

_NC = 2
_NS = 16
_NW = _NC * _NS
_ROWS_PER_W = (B * HW) // _NW
_CHUNK = 128
_NCHUNK = _ROWS_PER_W // _CHUNK


def _phase1_body(z_ref, emb_ref, zn_ref, val_ref, idx_ref, tru_ref):
    j = pl.program_id(1)
    e = emb_ref[...]
    z = z_ref[...]
    g = lax.dot_general(e.astype(jnp.bfloat16), z.astype(jnp.bfloat16),
                        (((1,), (1,)), ((), ())),
                        preferred_element_type=jnp.float32)
    en = jnp.sum(e * e, axis=1, keepdims=True)
    dist = (zn_ref[0] + en) - 2.0 * g
    rows = lax.broadcasted_iota(jnp.int32, dist.shape, 0) + j * TN
    dist = jnp.where(rows < NUM_EMB, dist, jnp.float32(jnp.inf))
    bmin = jnp.min(dist, axis=0, keepdims=True)
    tie = jnp.where(dist == bmin, rows, jnp.int32(2147483647))
    barg = jnp.min(tie, axis=0, keepdims=True)
    bq = bmin.astype(jnp.bfloat16).astype(jnp.float32)

    @pl.when(j == 0)
    def _():
        val_ref[0] = bq
        idx_ref[0] = barg
        tru_ref[0] = bmin

    @pl.when(j != 0)
    def _():
        cur = val_ref[0]
        better = bmin < cur
        val_ref[0] = jnp.where(better, bq, cur)
        idx_ref[0] = jnp.where(better, barg, idx_ref[0])
        tru_ref[0] = jnp.where(better, bmin, tru_ref[0])


def _phase1(zf, emb_weight, zn3, interpret=False):
    return pl.pallas_call(
        _phase1_body,
        grid=(B, N_BLOCKS),
        in_specs=[
            pl.BlockSpec((HW, EMB_DIM), lambda b, j: (b, 0)),
            pl.BlockSpec((TN, EMB_DIM), lambda b, j: (j, 0)),
            pl.BlockSpec((1, 1, HW), lambda b, j: (b, 0, 0)),
        ],
        out_specs=[
            pl.BlockSpec((1, 1, HW), lambda b, j: (b, 0, 0)),
            pl.BlockSpec((1, 1, HW), lambda b, j: (b, 0, 0)),
            pl.BlockSpec((1, 1, HW), lambda b, j: (b, 0, 0)),
        ],
        out_shape=[
            jax.ShapeDtypeStruct((B, 1, HW), jnp.float32),
            jax.ShapeDtypeStruct((B, 1, HW), jnp.int32),
            jax.ShapeDtypeStruct((B, 1, HW), jnp.float32),
        ],
        compiler_params=pltpu.CompilerParams(
            dimension_semantics=("parallel", "arbitrary")),
        interpret=interpret,
    )(zf, emb_weight, zn3)


def _sc_gather_body(table_hbm, idx_hbm, out_hbm, idx_v, rows_v, sem):
    wid = lax.axis_index("s") * _NC + lax.axis_index("c")
    for k in range(_NCHUNK):
        base = wid * _ROWS_PER_W + k * _CHUNK
        pltpu.sync_copy(idx_hbm.at[pl.ds(base, _CHUNK)], idx_v)
        pltpu.async_copy(table_hbm.at[idx_v], rows_v, sem).wait()
        pltpu.sync_copy(rows_v, out_hbm.at[pl.ds(base, _CHUNK)])


@functools.cache
def _make_sc_gather():
    return functools.partial(
        pl.kernel,
        mesh=plsc.VectorSubcoreMesh(core_axis_name="c", subcore_axis_name="s"),
        out_type=jax.ShapeDtypeStruct((B * HW, EMB_DIM), jnp.float32),
        scratch_types=[
            pltpu.VMEM((_CHUNK,), jnp.int32),
            pltpu.VMEM((_CHUNK, EMB_DIM), jnp.float32),
            pltpu.SemaphoreType.DMA,
        ],
    )(_sc_gather_body)


def kernel(z_e, emb_weight):
    b, c, h, w = z_e.shape
    z_flat = jnp.transpose(z_e, (0, 2, 3, 1)).reshape(b * h * w, c)
    zn = jnp.sum(z_flat ** 2, axis=-1, keepdims=True)
    zn3 = zn.reshape(b, 1, h * w)

    val3, idx3, tru3 = _phase1(z_flat, emb_weight, zn3)
    enc_idx = idx3.reshape(b * h * w)

    m = jnp.sum(tru3) / (b * h * w * c)
    loss = m + 0.25 * m

    zq_flat = _make_sc_gather()(emb_weight, enc_idx)
    z_q_st = jnp.transpose(zq_flat.reshape(b, h, w, c), (0, 3, 1, 2))
    return (z_q_st, loss, enc_idx)


# speedup vs baseline: 1.7991x; 1.0577x over previous
import functools

import jax
import jax.numpy as jnp
from jax import lax
from jax.experimental import pallas as pl
from jax.experimental.pallas import tpu as pltpu
from jax.experimental.pallas import tpu_sc as plsc

NUM_EMB = 8192
EMB_DIM = 256
B = 16
HW = 1024
TN = 2736
N_BLOCKS = 3

_NC = 2
_NS = 16
_NW = _NC * _NS
_ROWS_PER_W = (B * HW) // _NW
_CHUNK = 128
_NCHUNK = _ROWS_PER_W // _CHUNK


def _phase1_body(z_ref, emb_ref, zn_ref, val_ref, idx_ref, tru_ref):
    j = pl.program_id(1)
    e2 = emb_ref[...]
    z = z_ref[...]
    g2 = lax.dot_general(e2.astype(jnp.bfloat16), z.astype(jnp.bfloat16),
                         (((1,), (1,)), ((), ())),
                         preferred_element_type=jnp.float32)
    en = 0.25 * jnp.sum(e2 * e2, axis=1, keepdims=True)
    dist = (zn_ref[0] + en) + g2
    rows = lax.broadcasted_iota(jnp.int32, dist.shape, 0) + j * TN
    bmin = jnp.min(dist, axis=0, keepdims=True)
    tie = jnp.where(dist == bmin, rows, jnp.int32(2147483647))
    barg = jnp.min(tie, axis=0, keepdims=True)
    bq = bmin.astype(jnp.bfloat16).astype(jnp.float32)

    @pl.when(j == 0)
    def _():
        val_ref[0] = bq
        idx_ref[0] = barg
        tru_ref[0] = bmin

    @pl.when(j != 0)
    def _():
        cur = val_ref[0]
        better = bmin < cur
        val_ref[0] = jnp.where(better, bq, cur)
        idx_ref[0] = jnp.where(better, barg, idx_ref[0])
        tru_ref[0] = jnp.where(better, bmin, tru_ref[0])


def _phase1(zf, emb_weight, zn3, interpret=False):
    return pl.pallas_call(
        _phase1_body,
        grid=(B, N_BLOCKS),
        in_specs=[
            pl.BlockSpec((HW, EMB_DIM), lambda b, j: (b, 0)),
            pl.BlockSpec((TN, EMB_DIM), lambda b, j: (j, 0)),
            pl.BlockSpec((1, 1, HW), lambda b, j: (b, 0, 0)),
        ],
        out_specs=[
            pl.BlockSpec((1, 1, HW), lambda b, j: (b, 0, 0)),
            pl.BlockSpec((1, 1, HW), lambda b, j: (b, 0, 0)),
            pl.BlockSpec((1, 1, HW), lambda b, j: (b, 0, 0)),
        ],
        out_shape=[
            jax.ShapeDtypeStruct((B, 1, HW), jnp.float32),
            jax.ShapeDtypeStruct((B, 1, HW), jnp.int32),
            jax.ShapeDtypeStruct((B, 1, HW), jnp.float32),
        ],
        compiler_params=pltpu.CompilerParams(
            dimension_semantics=("parallel", "arbitrary")),
        interpret=interpret,
    )(zf, emb_weight, zn3)


def _sc_gather_body(table_hbm, idx_hbm, out_hbm, idx_v, rows_v, sem):
    wid = lax.axis_index("s") * _NC + lax.axis_index("c")
    for k in range(_NCHUNK):
        base = wid * _ROWS_PER_W + k * _CHUNK
        pltpu.sync_copy(idx_hbm.at[pl.ds(base, _CHUNK)], idx_v)
        pltpu.async_copy(table_hbm.at[idx_v], rows_v, sem).wait()
        pltpu.sync_copy(rows_v, out_hbm.at[pl.ds(base, _CHUNK)])


@functools.cache
def _make_sc_gather():
    return functools.partial(
        pl.kernel,
        mesh=plsc.VectorSubcoreMesh(core_axis_name="c", subcore_axis_name="s"),
        out_type=jax.ShapeDtypeStruct((B * HW, EMB_DIM), jnp.float32),
        scratch_types=[
            pltpu.VMEM((_CHUNK,), jnp.int32),
            pltpu.VMEM((_CHUNK, EMB_DIM), jnp.float32),
            pltpu.SemaphoreType.DMA,
        ],
    )(_sc_gather_body)


def kernel(z_e, emb_weight):
    b, c, h, w = z_e.shape
    z_flat = jnp.transpose(z_e, (0, 2, 3, 1)).reshape(b * h * w, c)
    zn = jnp.sum(z_flat ** 2, axis=-1, keepdims=True)
    zn3 = zn.reshape(b, 1, h * w)

    emb2 = jnp.concatenate(
        [emb_weight * jnp.float32(-2.0),
         jnp.full((N_BLOCKS * TN - NUM_EMB, c), -200.0, jnp.float32)], axis=0)

    val3, idx3, tru3 = _phase1(z_flat, emb2, zn3)
    enc_idx = idx3.reshape(b * h * w)

    m = jnp.sum(tru3) / (b * h * w * c)
    loss = m + 0.25 * m

    zq_flat = _make_sc_gather()(emb_weight, enc_idx)
    z_q_st = jnp.transpose(zq_flat.reshape(b, h, w, c), (0, 3, 1, 2))
    return (z_q_st, loss, enc_idx)
